# baseline (speedup 1.0000x reference)
import functools

import jax
import numpy as np
import jax.numpy as jnp
from jax import lax
from jax.experimental import pallas as pl
from jax.experimental.pallas import tpu as pltpu
from jax.experimental.pallas import tpu_sc as plsc

_TARGET_H = 8.0
_NBG = 4096
_B = 1024
_D = 16
_K = 100000

_CBLK = 2048
_NCB = (_K + _CBLK - 1) // _CBLK
_SPB = 128
_NS = _NCB * _SPB
_SRANK = 512
_W = 12288
_KSC = 99968
_TAILC = _K - _KSC
_CHL = 2944
_NFC = 33
_TCH = _KSC - _NFC * _CHL
_SENT = -1.0e30

_TOPBIT = np.uint32(0x80000000)


def _keyify(v):
    b = lax.bitcast_convert_type(v, jnp.uint32)
    return jnp.where(b >= _TOPBIT, ~b, b | _TOPBIT)


def _unkeyify(k):
    b = jnp.where(k >= _TOPBIT, k ^ _TOPBIT, ~k)
    return lax.bitcast_convert_type(b, jnp.float32)


def _sims_kernel(pts_ref, bank_ref, out_ref, smp_ref):
    pts = pts_ref[...]
    pn = pts / jnp.sqrt(jnp.sum(pts * pts, axis=1, keepdims=True))
    bk = bank_ref[...]
    bn = bk / jnp.sqrt(jnp.sum(bk * bk, axis=1, keepdims=True))
    blk = lax.dot_general(pn, bn, (((1,), (1,)), ((), ())),
                          preferred_element_type=jnp.float32)
    out_ref[...] = blk
    smp_ref[...] = blk[:, :_SPB]


def _compute_sims(points, memory_bank):
    return pl.pallas_call(
        _sims_kernel,
        grid=(_NCB,),
        in_specs=[
            pl.BlockSpec((_B, _D), lambda i: (0, 0)),
            pl.BlockSpec((_CBLK, _D), lambda i: (i, 0)),
        ],
        out_specs=[
            pl.BlockSpec((_B, _CBLK), lambda i: (0, i)),
            pl.BlockSpec((_B, _SPB), lambda i: (0, i)),
        ],
        out_shape=[
            jax.ShapeDtypeStruct((_B, _K), jnp.float32),
            jax.ShapeDtypeStruct((_B, _NS), jnp.float32),
        ],
    )(points, memory_bank)


def _sample_bisect_kernel(smp_ref, tau_ref):
    key = _keyify(smp_ref[...])
    lo = jnp.zeros((_B, 1), jnp.uint32)
    for bit in range(31, -1, -1):
        cand = lo | np.uint32(1 << bit)
        cnt = jnp.sum((key >= cand).astype(jnp.int32), axis=1, keepdims=True)
        lo = jnp.where(cnt >= _SRANK, cand, lo)
    tau = _unkeyify(lo)
    tau_ref[...] = jnp.broadcast_to(tau, (_B, 16))


def _sample_bisect(sample):
    return pl.pallas_call(
        _sample_bisect_kernel,
        in_specs=[pl.BlockSpec((_B, _NS), lambda: (0, 0))],
        out_specs=pl.BlockSpec((_B, 16), lambda: (0, 0)),
        out_shape=jax.ShapeDtypeStruct((_B, 16), jnp.float32),
    )(sample)


def _sc_compact(sims_hbm, tau_hbm, out_hbm, cbuf, stage, tauv8):
    wid = lax.axis_index("s") * 2 + lax.axis_index("c")

    def group_body(gi, _):
        g = wid * 4 + gi
        base = g * 8

        def fill(j, u):
            stage[pl.ds(j * 16, 16)] = jnp.full((16,), _SENT, jnp.float32)
            return u
        lax.fori_loop(0, 8 * _W // 16, fill, 0)

        pltpu.sync_copy(tau_hbm.at[pl.ds(base, 8)], tauv8)

        def compact_rows(nvr, offs):
            def vbody(j, os_):
                new = []
                for x in range(8):
                    v = cbuf[x, pl.ds(j * 16, 16)].reshape(16)
                    m = v >= tauv8[x, :].reshape(16)
                    cs = plsc.cumsum(m.astype(jnp.int32))
                    idx = (x * _W + os_[x] - 1) + cs
                    plsc.store_scatter(stage, [idx], v, mask=m)
                    new.append(jnp.minimum(os_[x] + cs[15], _W - 16))
                return tuple(new)
            return lax.fori_loop(0, nvr, vbody, offs)

        def chunk_body(c, offs):
            pltpu.sync_copy(
                sims_hbm.at[pl.ds(base, 8), pl.ds(c * _CHL, _CHL)], cbuf)
            return compact_rows(_CHL // 16, offs)

        offs = lax.fori_loop(0, _NFC, chunk_body, (jnp.int32(0),) * 8)
        pltpu.sync_copy(
            sims_hbm.at[pl.ds(base, 8), pl.ds(_NFC * _CHL, _TCH)],
            cbuf.at[:, pl.ds(0, _TCH)])
        compact_rows(_TCH // 16, offs)

        for x in range(8):
            pltpu.sync_copy(stage.at[pl.ds(x * _W, _W)],
                            out_hbm.at[pl.ds((base + x) * _W, _W)])
        return _

    lax.fori_loop(0, 4, group_body, 0)


def _compact(sims, tau16):
    mesh = plsc.VectorSubcoreMesh(core_axis_name="c", subcore_axis_name="s")
    f = functools.partial(
        pl.kernel,
        out_type=jax.ShapeDtypeStruct((_B * _W,), jnp.float32),
        mesh=mesh,
        compiler_params=pltpu.CompilerParams(needs_layout_passes=False),
        scratch_types=[
            pltpu.VMEM((8, _CHL), jnp.float32),
            pltpu.VMEM((8 * _W,), jnp.float32),
            pltpu.VMEM((8, 16), jnp.float32),
        ],
    )(_sc_compact)
    return f(sims, tau16).reshape(_B, _W)


_RB = 128


def _final_kernel(buf_ref, sims_ref, tail_ref, pidx_ref, c_ref, e_ref, l_ref):
    tail = tail_ref[...]
    tcol = lax.broadcasted_iota(jnp.int32, (_RB, 128), 1)
    tail = jnp.where(tcol < _TAILC, tail, _SENT)
    s = jnp.concatenate([buf_ref[...], tail], axis=1)
    p = pidx_ref[...]
    key = _keyify(s)

    lo = jnp.zeros((_RB, 1), jnp.uint32)
    for bit in range(31, -1, -1):
        cand = lo | np.uint32(1 << bit)
        cnt = jnp.sum((key >= cand).astype(jnp.int32), axis=1, keepdims=True)
        lo = jnp.where(cnt >= _NBG, cand, lo)
    taukey = lo

    tgt = p + 1
    lo2 = jnp.zeros((_RB, 1), jnp.uint32)
    for bit in range(31, -1, -1):
        cand = lo2 | np.uint32(1 << bit)
        cnt = jnp.sum((key >= cand).astype(jnp.int32), axis=1, keepdims=True)
        cnt = jnp.minimum(cnt, _NBG)
        lo2 = jnp.where(cnt >= tgt, cand, lo2)

    tau = _unkeyify(taukey)
    vp = _unkeyify(lo2)
    gt = key > taukey
    kgt = jnp.sum(gt.astype(jnp.int32), axis=1, keepdims=True)
    npad = (_NBG - kgt).astype(jnp.float32)
    sm = jnp.where(gt, s, _SENT)

    s4 = sims_ref[...]
    iot = lax.broadcasted_iota(jnp.int32, (_RB, _NBG), 1)
    ps = jnp.sum(jnp.where(iot == p, s4, 0.0), axis=1, keepdims=True)

    def entropy_at(t):
        F = jnp.exp(sm / t)
        Ft = jnp.exp(tau / t)
        Fp = jnp.exp(vp / t)
        Z = jnp.sum(F, axis=1, keepdims=True) + npad * Ft - Fp
        r = F / Z
        ent = -jnp.sum(r * jnp.log(r + 1e-7), axis=1, keepdims=True)
        rt = Ft / Z
        rp = Fp / Z
        ent = ent - npad * (rt * jnp.log(rt + 1e-7)) + rp * jnp.log(rp + 1e-7)
        return ent

    t = jnp.full((_RB, 1), 5.0, jnp.float32)
    scale = 2.5
    for _ in range(13):
        ent = entropy_at(t)
        ind = 2.0 * (ent < _TARGET_H).astype(jnp.float32) - 1.0
        t = t + scale * ind
        scale = scale * 0.5
    ent = entropy_at(t)

    den = (jnp.sum(jnp.exp(sm / t - 1.0 / t), axis=1, keepdims=True)
           + npad * jnp.exp(tau / t - 1.0 / t))
    cp = jnp.exp(ps / t - 1.0 / t) / den
    c_ref[...] = t
    e_ref[...] = ent
    l_ref[...] = jnp.log(cp + 1e-7)


def _finalize(buf, sims, point_indices):
    nrb = _B // _RB
    return pl.pallas_call(
        _final_kernel,
        grid=(nrb,),
        in_specs=[
            pl.BlockSpec((_RB, _W), lambda i: (i, 0)),
            pl.BlockSpec((_RB, _NBG), lambda i: (i, 0)),
            pl.BlockSpec((_RB, 128), lambda i: (i, _KSC // 128)),
            pl.BlockSpec((_RB, 1), lambda i: (i, 0)),
        ],
        out_specs=[
            pl.BlockSpec((_RB, 1), lambda i: (i, 0)),
            pl.BlockSpec((_RB, 1), lambda i: (i, 0)),
            pl.BlockSpec((_RB, 1), lambda i: (i, 0)),
        ],
        out_shape=[
            jax.ShapeDtypeStruct((_B, 1), jnp.float32),
            jax.ShapeDtypeStruct((_B, 1), jnp.float32),
            jax.ShapeDtypeStruct((_B, 1), jnp.float32),
        ],
    )(buf, sims, sims, point_indices.reshape(_B, 1).astype(jnp.int32))


def kernel(points, point_indices, memory_bank):
    sims, sample = _compute_sims(points, memory_bank)
    tau16 = _sample_bisect(sample)
    buf = _compact(sims, tau16)
    centers, entropy, logterm = _finalize(buf, sims, point_indices)
    loss = -jnp.mean(logterm)
    return loss, sims, jnp.mean(centers), jnp.mean(entropy)


# speedup vs baseline: 1.0821x; 1.0821x over previous
import functools

import jax
import numpy as np
import jax.numpy as jnp
from jax import lax
from jax.experimental import pallas as pl
from jax.experimental.pallas import tpu as pltpu
from jax.experimental.pallas import tpu_sc as plsc

_TARGET_H = 8.0
_NBG = 4096
_B = 1024
_D = 16
_K = 100000

_CBLK = 2048
_NCB = (_K + _CBLK - 1) // _CBLK
_SPB = 128
_NS = _NCB * _SPB
_SRANK = 512
_W = 12288
_KSC = 99968
_TAILC = _K - _KSC
_CHL = 1408
_NFC = 71
_SENT = -1.0e30

_TOPBIT = np.uint32(0x80000000)


def _keyify(v):
    b = lax.bitcast_convert_type(v, jnp.uint32)
    return jnp.where(b >= _TOPBIT, ~b, b | _TOPBIT)


def _unkeyify(k):
    b = jnp.where(k >= _TOPBIT, k ^ _TOPBIT, ~k)
    return lax.bitcast_convert_type(b, jnp.float32)


def _sims_kernel(pts_ref, bank_ref, out_ref, smp_ref):
    pts = pts_ref[...]
    pn = pts / jnp.sqrt(jnp.sum(pts * pts, axis=1, keepdims=True))
    bk = bank_ref[...]
    bn = bk / jnp.sqrt(jnp.sum(bk * bk, axis=1, keepdims=True))
    blk = lax.dot_general(pn, bn, (((1,), (1,)), ((), ())),
                          preferred_element_type=jnp.float32)
    out_ref[...] = blk
    smp_ref[...] = blk[:, :_SPB]


def _compute_sims(points, memory_bank):
    return pl.pallas_call(
        _sims_kernel,
        grid=(_NCB,),
        in_specs=[
            pl.BlockSpec((_B, _D), lambda i: (0, 0)),
            pl.BlockSpec((_CBLK, _D), lambda i: (i, 0)),
        ],
        out_specs=[
            pl.BlockSpec((_B, _CBLK), lambda i: (0, i)),
            pl.BlockSpec((_B, _SPB), lambda i: (0, i)),
        ],
        out_shape=[
            jax.ShapeDtypeStruct((_B, _K), jnp.float32),
            jax.ShapeDtypeStruct((_B, _NS), jnp.float32),
        ],
    )(points, memory_bank)


def _sample_bisect_kernel(smp_ref, tau_ref):
    key = _keyify(smp_ref[...])
    lo = jnp.zeros((_B, 1), jnp.uint32)
    for bit in range(31, -1, -1):
        cand = lo | np.uint32(1 << bit)
        cnt = jnp.sum((key >= cand).astype(jnp.int32), axis=1, keepdims=True)
        lo = jnp.where(cnt >= _SRANK, cand, lo)
    tau = _unkeyify(lo)
    tau_ref[...] = jnp.broadcast_to(tau, (_B, 16))


def _sample_bisect(sample):
    return pl.pallas_call(
        _sample_bisect_kernel,
        in_specs=[pl.BlockSpec((_B, _NS), lambda: (0, 0))],
        out_specs=pl.BlockSpec((_B, 16), lambda: (0, 0)),
        out_shape=jax.ShapeDtypeStruct((_B, 16), jnp.float32),
    )(sample)


def _sc_compact(sims_hbm, tau_hbm, out_hbm, cbuf, cbufb, stage, tauv8,
                sema, semb):
    wid = lax.axis_index("s") * 2 + lax.axis_index("c")

    def group_body(gi, _):
        g = wid * 4 + gi
        base = g * 8

        def fill(j, u):
            stage[pl.ds(j * 16, 16)] = jnp.full((16,), _SENT, jnp.float32)
            return u
        lax.fori_loop(0, 8 * _W // 16, fill, 0)

        pltpu.sync_copy(tau_hbm.at[pl.ds(base, 8)], tauv8)
        tvals = [tauv8[x, :].reshape(16) for x in range(8)]

        def compact_rows(cb, nvr, offs):
            def vbody(j, os_):
                new = []
                for x in range(8):
                    v = cb[x, pl.ds(j * 16, 16)].reshape(16)
                    m = v >= tvals[x]
                    cs = plsc.cumsum(m.astype(jnp.int32))
                    idx = (x * _W + os_[x] - 1) + cs
                    plsc.store_scatter(stage, [idx], v, mask=m)
                    new.append(jnp.minimum(os_[x] + cs[15], _W - 16))
                return tuple(new)
            return lax.fori_loop(0, nvr, vbody, offs)

        def issue(c, buf, sem):
            return pltpu.async_copy(
                sims_hbm.at[pl.ds(base, 8), pl.ds(c * _CHL, _CHL)], buf, sem)

        def wait(buf, sem):
            pltpu.make_async_copy(
                sims_hbm.at[pl.ds(0, 8), pl.ds(0, _CHL)], buf, sem).wait()

        issue(0, cbuf, sema)

        def dbl_body(c2, offs):
            issue(2 * c2 + 1, cbufb, semb)
            wait(cbuf, sema)
            offs = compact_rows(cbuf, _CHL // 16, offs)
            issue(2 * c2 + 2, cbuf, sema)
            wait(cbufb, semb)
            return compact_rows(cbufb, _CHL // 16, offs)

        offs = lax.fori_loop(0, (_NFC - 1) // 2, dbl_body, (jnp.int32(0),) * 8)
        wait(cbuf, sema)
        compact_rows(cbuf, _CHL // 16, offs)

        for x in range(8):
            pltpu.sync_copy(stage.at[pl.ds(x * _W, _W)],
                            out_hbm.at[pl.ds((base + x) * _W, _W)])
        return _

    lax.fori_loop(0, 4, group_body, 0)


def _compact(sims, tau16):
    mesh = plsc.VectorSubcoreMesh(core_axis_name="c", subcore_axis_name="s")
    f = functools.partial(
        pl.kernel,
        out_type=jax.ShapeDtypeStruct((_B * _W,), jnp.float32),
        mesh=mesh,
        compiler_params=pltpu.CompilerParams(needs_layout_passes=False),
        scratch_types=[
            pltpu.VMEM((8, _CHL), jnp.float32),
            pltpu.VMEM((8, _CHL), jnp.float32),
            pltpu.VMEM((8 * _W,), jnp.float32),
            pltpu.VMEM((8, 16), jnp.float32),
            pltpu.SemaphoreType.DMA,
            pltpu.SemaphoreType.DMA,
        ],
    )(_sc_compact)
    return f(sims, tau16).reshape(_B, _W)


_RB = 128


def _final_kernel(buf_ref, sims_ref, tail_ref, pidx_ref, c_ref, e_ref, l_ref):
    tail = tail_ref[...]
    tcol = lax.broadcasted_iota(jnp.int32, (_RB, 128), 1)
    tail = jnp.where(tcol < _TAILC, tail, _SENT)
    s = jnp.concatenate([buf_ref[...], tail], axis=1)
    p = pidx_ref[...]
    key = _keyify(s)

    lo = jnp.zeros((_RB, 1), jnp.uint32)
    for bit in range(31, -1, -1):
        cand = lo | np.uint32(1 << bit)
        cnt = jnp.sum((key >= cand).astype(jnp.int32), axis=1, keepdims=True)
        lo = jnp.where(cnt >= _NBG, cand, lo)
    taukey = lo

    tgt = p + 1
    lo2 = jnp.zeros((_RB, 1), jnp.uint32)
    for bit in range(31, -1, -1):
        cand = lo2 | np.uint32(1 << bit)
        cnt = jnp.sum((key >= cand).astype(jnp.int32), axis=1, keepdims=True)
        cnt = jnp.minimum(cnt, _NBG)
        lo2 = jnp.where(cnt >= tgt, cand, lo2)

    tau = _unkeyify(taukey)
    vp = _unkeyify(lo2)
    gt = key > taukey
    kgt = jnp.sum(gt.astype(jnp.int32), axis=1, keepdims=True)
    npad = (_NBG - kgt).astype(jnp.float32)
    sm = jnp.where(gt, s, _SENT)

    s4 = sims_ref[...]
    iot = lax.broadcasted_iota(jnp.int32, (_RB, _NBG), 1)
    ps = jnp.sum(jnp.where(iot == p, s4, 0.0), axis=1, keepdims=True)

    def entropy_at(t):
        F = jnp.exp(sm / t)
        Ft = jnp.exp(tau / t)
        Fp = jnp.exp(vp / t)
        Z = jnp.sum(F, axis=1, keepdims=True) + npad * Ft - Fp
        r = F / Z
        ent = -jnp.sum(r * jnp.log(r + 1e-7), axis=1, keepdims=True)
        rt = Ft / Z
        rp = Fp / Z
        ent = ent - npad * (rt * jnp.log(rt + 1e-7)) + rp * jnp.log(rp + 1e-7)
        return ent

    t = jnp.full((_RB, 1), 5.0, jnp.float32)
    scale = 2.5
    for _ in range(13):
        ent = entropy_at(t)
        ind = 2.0 * (ent < _TARGET_H).astype(jnp.float32) - 1.0
        t = t + scale * ind
        scale = scale * 0.5
    ent = entropy_at(t)

    den = (jnp.sum(jnp.exp(sm / t - 1.0 / t), axis=1, keepdims=True)
           + npad * jnp.exp(tau / t - 1.0 / t))
    cp = jnp.exp(ps / t - 1.0 / t) / den
    c_ref[...] = t
    e_ref[...] = ent
    l_ref[...] = jnp.log(cp + 1e-7)


def _finalize(buf, sims, point_indices):
    nrb = _B // _RB
    return pl.pallas_call(
        _final_kernel,
        grid=(nrb,),
        in_specs=[
            pl.BlockSpec((_RB, _W), lambda i: (i, 0)),
            pl.BlockSpec((_RB, _NBG), lambda i: (i, 0)),
            pl.BlockSpec((_RB, 128), lambda i: (i, _KSC // 128)),
            pl.BlockSpec((_RB, 1), lambda i: (i, 0)),
        ],
        out_specs=[
            pl.BlockSpec((_RB, 1), lambda i: (i, 0)),
            pl.BlockSpec((_RB, 1), lambda i: (i, 0)),
            pl.BlockSpec((_RB, 1), lambda i: (i, 0)),
        ],
        out_shape=[
            jax.ShapeDtypeStruct((_B, 1), jnp.float32),
            jax.ShapeDtypeStruct((_B, 1), jnp.float32),
            jax.ShapeDtypeStruct((_B, 1), jnp.float32),
        ],
    )(buf, sims, sims, point_indices.reshape(_B, 1).astype(jnp.int32))


def kernel(points, point_indices, memory_bank):
    sims, sample = _compute_sims(points, memory_bank)
    tau16 = _sample_bisect(sample)
    buf = _compact(sims, tau16)
    centers, entropy, logterm = _finalize(buf, sims, point_indices)
    loss = -jnp.mean(logterm)
    return loss, sims, jnp.mean(centers), jnp.mean(entropy)


# speedup vs baseline: 1.1460x; 1.0590x over previous
import functools

import jax
import numpy as np
import jax.numpy as jnp
from jax import lax
from jax.experimental import pallas as pl
from jax.experimental.pallas import tpu as pltpu
from jax.experimental.pallas import tpu_sc as plsc

_TARGET_H = 8.0
_NBG = 4096
_B = 1024
_D = 16
_K = 100000

_CBLK = 2048
_NCB = (_K + _CBLK - 1) // _CBLK
_SPB = 128
_NS = _NCB * _SPB
_SRANK = 384
_W = 9216
_KSC = 99968
_TAILC = _K - _KSC
_CHL = 1408
_NFC = 71
_SENT = -1.0e30

_TOPBIT = np.uint32(0x80000000)


def _keyify(v):
    b = lax.bitcast_convert_type(v, jnp.uint32)
    return jnp.where(b >= _TOPBIT, ~b, b | _TOPBIT)


def _unkeyify(k):
    b = jnp.where(k >= _TOPBIT, k ^ _TOPBIT, ~k)
    return lax.bitcast_convert_type(b, jnp.float32)


def _sims_kernel(pts_ref, bank_ref, out_ref, smp_ref):
    pts = pts_ref[...]
    pn = pts / jnp.sqrt(jnp.sum(pts * pts, axis=1, keepdims=True))
    bk = bank_ref[...]
    bn = bk / jnp.sqrt(jnp.sum(bk * bk, axis=1, keepdims=True))
    blk = lax.dot_general(pn, bn, (((1,), (1,)), ((), ())),
                          preferred_element_type=jnp.float32)
    out_ref[...] = blk
    smp_ref[...] = blk[:, :_SPB]


def _compute_sims(points, memory_bank):
    return pl.pallas_call(
        _sims_kernel,
        grid=(_NCB,),
        in_specs=[
            pl.BlockSpec((_B, _D), lambda i: (0, 0)),
            pl.BlockSpec((_CBLK, _D), lambda i: (i, 0)),
        ],
        out_specs=[
            pl.BlockSpec((_B, _CBLK), lambda i: (0, i)),
            pl.BlockSpec((_B, _SPB), lambda i: (0, i)),
        ],
        out_shape=[
            jax.ShapeDtypeStruct((_B, _K), jnp.float32),
            jax.ShapeDtypeStruct((_B, _NS), jnp.float32),
        ],
    )(points, memory_bank)


def _sample_bisect_kernel(smp_ref, tau_ref):
    key = _keyify(smp_ref[...])
    lo = jnp.zeros((_B, 1), jnp.uint32)
    for bit in range(31, -1, -1):
        cand = lo | np.uint32(1 << bit)
        cnt = jnp.sum((key >= cand).astype(jnp.int32), axis=1, keepdims=True)
        lo = jnp.where(cnt >= _SRANK, cand, lo)
    tau = _unkeyify(lo)
    tau_ref[...] = jnp.broadcast_to(tau, (_B, 16))


def _sample_bisect(sample):
    return pl.pallas_call(
        _sample_bisect_kernel,
        in_specs=[pl.BlockSpec((_B, _NS), lambda: (0, 0))],
        out_specs=pl.BlockSpec((_B, 16), lambda: (0, 0)),
        out_shape=jax.ShapeDtypeStruct((_B, 16), jnp.float32),
    )(sample)


def _sc_compact(sims_hbm, tau_hbm, out_hbm, cbuf, cbufb, stage, tauv8,
                sema, semb):
    wid = lax.axis_index("s") * 2 + lax.axis_index("c")

    def group_body(gi, _):
        g = wid * 4 + gi
        base = g * 8

        def fill(j, u):
            stage[pl.ds(j * 16, 16)] = jnp.full((16,), _SENT, jnp.float32)
            return u
        lax.fori_loop(0, 8 * _W // 16, fill, 0)

        pltpu.sync_copy(tau_hbm.at[pl.ds(base, 8)], tauv8)
        tvals = [tauv8[x, :].reshape(16) for x in range(8)]

        def compact_rows(cb, nvr, offs):
            def vbody(j, os_):
                new = []
                for x in range(8):
                    v = cb[x, pl.ds(j * 16, 16)].reshape(16)
                    m = v >= tvals[x]
                    cs = plsc.cumsum(m.astype(jnp.int32))
                    idx = (x * _W + os_[x] - 1) + cs
                    plsc.store_scatter(stage, [idx], v, mask=m)
                    new.append(jnp.minimum(os_[x] + cs[15], _W - 16))
                return tuple(new)
            return lax.fori_loop(0, nvr, vbody, offs)

        def issue(c, buf, sem):
            return pltpu.async_copy(
                sims_hbm.at[pl.ds(base, 8), pl.ds(c * _CHL, _CHL)], buf, sem)

        def wait(buf, sem):
            pltpu.make_async_copy(
                sims_hbm.at[pl.ds(0, 8), pl.ds(0, _CHL)], buf, sem).wait()

        issue(0, cbuf, sema)

        def dbl_body(c2, offs):
            issue(2 * c2 + 1, cbufb, semb)
            wait(cbuf, sema)
            offs = compact_rows(cbuf, _CHL // 16, offs)
            issue(2 * c2 + 2, cbuf, sema)
            wait(cbufb, semb)
            return compact_rows(cbufb, _CHL // 16, offs)

        offs = lax.fori_loop(0, (_NFC - 1) // 2, dbl_body, (jnp.int32(0),) * 8)
        wait(cbuf, sema)
        compact_rows(cbuf, _CHL // 16, offs)

        for x in range(8):
            pltpu.sync_copy(stage.at[pl.ds(x * _W, _W)],
                            out_hbm.at[pl.ds((base + x) * _W, _W)])
        return _

    lax.fori_loop(0, 4, group_body, 0)


def _compact(sims, tau16):
    mesh = plsc.VectorSubcoreMesh(core_axis_name="c", subcore_axis_name="s")
    f = functools.partial(
        pl.kernel,
        out_type=jax.ShapeDtypeStruct((_B * _W,), jnp.float32),
        mesh=mesh,
        compiler_params=pltpu.CompilerParams(needs_layout_passes=False),
        scratch_types=[
            pltpu.VMEM((8, _CHL), jnp.float32),
            pltpu.VMEM((8, _CHL), jnp.float32),
            pltpu.VMEM((8 * _W,), jnp.float32),
            pltpu.VMEM((8, 16), jnp.float32),
            pltpu.SemaphoreType.DMA,
            pltpu.SemaphoreType.DMA,
        ],
    )(_sc_compact)
    return f(sims, tau16).reshape(_B, _W)


_RB = 128


def _final_kernel(buf_ref, sims_ref, tail_ref, pidx_ref, c_ref, e_ref, l_ref):
    tail = tail_ref[...]
    tcol = lax.broadcasted_iota(jnp.int32, (_RB, 128), 1)
    tail = jnp.where(tcol < _TAILC, tail, _SENT)
    s = jnp.concatenate([buf_ref[...], tail], axis=1)
    p = pidx_ref[...]
    key = _keyify(s)

    lo = jnp.zeros((_RB, 1), jnp.uint32)
    for bit in range(31, -1, -1):
        cand = lo | np.uint32(1 << bit)
        cnt = jnp.sum((key >= cand).astype(jnp.int32), axis=1, keepdims=True)
        lo = jnp.where(cnt >= _NBG, cand, lo)
    taukey = lo

    tgt = p + 1
    lo2 = jnp.zeros((_RB, 1), jnp.uint32)
    for bit in range(31, -1, -1):
        cand = lo2 | np.uint32(1 << bit)
        cnt = jnp.sum((key >= cand).astype(jnp.int32), axis=1, keepdims=True)
        cnt = jnp.minimum(cnt, _NBG)
        lo2 = jnp.where(cnt >= tgt, cand, lo2)

    tau = _unkeyify(taukey)
    vp = _unkeyify(lo2)
    gt = key > taukey
    kgt = jnp.sum(gt.astype(jnp.int32), axis=1, keepdims=True)
    npad = (_NBG - kgt).astype(jnp.float32)
    sm = jnp.where(gt, s, _SENT)

    s4 = sims_ref[...]
    iot = lax.broadcasted_iota(jnp.int32, (_RB, _NBG), 1)
    ps = jnp.sum(jnp.where(iot == p, s4, 0.0), axis=1, keepdims=True)

    def entropy_at(t):
        F = jnp.exp(sm / t)
        Ft = jnp.exp(tau / t)
        Fp = jnp.exp(vp / t)
        Z = jnp.sum(F, axis=1, keepdims=True) + npad * Ft - Fp
        r = F / Z
        ent = -jnp.sum(r * jnp.log(r + 1e-7), axis=1, keepdims=True)
        rt = Ft / Z
        rp = Fp / Z
        ent = ent - npad * (rt * jnp.log(rt + 1e-7)) + rp * jnp.log(rp + 1e-7)
        return ent

    t = jnp.full((_RB, 1), 5.0, jnp.float32)
    scale = 2.5
    for _ in range(13):
        ent = entropy_at(t)
        ind = 2.0 * (ent < _TARGET_H).astype(jnp.float32) - 1.0
        t = t + scale * ind
        scale = scale * 0.5
    ent = entropy_at(t)

    den = (jnp.sum(jnp.exp(sm / t - 1.0 / t), axis=1, keepdims=True)
           + npad * jnp.exp(tau / t - 1.0 / t))
    cp = jnp.exp(ps / t - 1.0 / t) / den
    c_ref[...] = t
    e_ref[...] = ent
    l_ref[...] = jnp.log(cp + 1e-7)


def _finalize(buf, sims, point_indices):
    nrb = _B // _RB
    return pl.pallas_call(
        _final_kernel,
        grid=(nrb,),
        in_specs=[
            pl.BlockSpec((_RB, _W), lambda i: (i, 0)),
            pl.BlockSpec((_RB, _NBG), lambda i: (i, 0)),
            pl.BlockSpec((_RB, 128), lambda i: (i, _KSC // 128)),
            pl.BlockSpec((_RB, 1), lambda i: (i, 0)),
        ],
        out_specs=[
            pl.BlockSpec((_RB, 1), lambda i: (i, 0)),
            pl.BlockSpec((_RB, 1), lambda i: (i, 0)),
            pl.BlockSpec((_RB, 1), lambda i: (i, 0)),
        ],
        out_shape=[
            jax.ShapeDtypeStruct((_B, 1), jnp.float32),
            jax.ShapeDtypeStruct((_B, 1), jnp.float32),
            jax.ShapeDtypeStruct((_B, 1), jnp.float32),
        ],
    )(buf, sims, sims, point_indices.reshape(_B, 1).astype(jnp.int32))


def kernel(points, point_indices, memory_bank):
    sims, sample = _compute_sims(points, memory_bank)
    tau16 = _sample_bisect(sample)
    buf = _compact(sims, tau16)
    centers, entropy, logterm = _finalize(buf, sims, point_indices)
    loss = -jnp.mean(logterm)
    return loss, sims, jnp.mean(centers), jnp.mean(entropy)


# speedup vs baseline: 1.1717x; 1.0224x over previous
import functools

import jax
import numpy as np
import jax.numpy as jnp
from jax import lax
from jax.experimental import pallas as pl
from jax.experimental.pallas import tpu as pltpu
from jax.experimental.pallas import tpu_sc as plsc

_TARGET_H = 8.0
_NBG = 4096
_B = 1024
_D = 16
_K = 100000

_CBLK = 2048
_NCB = (_K + _CBLK - 1) // _CBLK
_SPB = 128
_NS = _NCB * _SPB
_SRANK = 384
_W = 9216
_KSC = 99968
_TAILC = _K - _KSC
_CHL = 1408
_NFC = 71
_SENT = -1.0e30

_TOPBIT = np.uint32(0x80000000)


def _keyify(v):
    b = lax.bitcast_convert_type(v, jnp.uint32)
    return jnp.where(b >= _TOPBIT, ~b, b | _TOPBIT)


def _unkeyify(k):
    b = jnp.where(k >= _TOPBIT, k ^ _TOPBIT, ~k)
    return lax.bitcast_convert_type(b, jnp.float32)


def _sims_kernel(pts_ref, bank_ref, out_ref, smp_ref):
    pts = pts_ref[...]
    pn = pts / jnp.sqrt(jnp.sum(pts * pts, axis=1, keepdims=True))
    bk = bank_ref[...]
    bn = bk / jnp.sqrt(jnp.sum(bk * bk, axis=1, keepdims=True))
    blk = lax.dot_general(pn, bn, (((1,), (1,)), ((), ())),
                          preferred_element_type=jnp.float32)
    out_ref[...] = blk
    smp_ref[...] = blk[:, :_SPB]


def _compute_sims(points, memory_bank):
    return pl.pallas_call(
        _sims_kernel,
        grid=(_NCB,),
        in_specs=[
            pl.BlockSpec((_B, _D), lambda i: (0, 0)),
            pl.BlockSpec((_CBLK, _D), lambda i: (i, 0)),
        ],
        out_specs=[
            pl.BlockSpec((_B, _CBLK), lambda i: (0, i)),
            pl.BlockSpec((_B, _SPB), lambda i: (0, i)),
        ],
        out_shape=[
            jax.ShapeDtypeStruct((_B, _K), jnp.float32),
            jax.ShapeDtypeStruct((_B, _NS), jnp.float32),
        ],
    )(points, memory_bank)


def _sample_bisect_kernel(smp_ref, tau_ref):
    key = _keyify(smp_ref[...])
    lo = jnp.zeros((_B, 1), jnp.uint32)
    for bit in range(31, -1, -1):
        cand = lo | np.uint32(1 << bit)
        cnt = jnp.sum((key >= cand).astype(jnp.int32), axis=1, keepdims=True)
        lo = jnp.where(cnt >= _SRANK, cand, lo)
    tau = _unkeyify(lo)
    tau_ref[...] = jnp.broadcast_to(tau, (_B, 16))


def _sample_bisect(sample):
    return pl.pallas_call(
        _sample_bisect_kernel,
        in_specs=[pl.BlockSpec((_B, _NS), lambda: (0, 0))],
        out_specs=pl.BlockSpec((_B, 16), lambda: (0, 0)),
        out_shape=jax.ShapeDtypeStruct((_B, 16), jnp.float32),
    )(sample)


def _sc_compact(sims_hbm, tau_hbm, out_hbm, cbuf, cbufb, stage, tauv8,
                sema, semb):
    wid = lax.axis_index("s") * 2 + lax.axis_index("c")

    def group_body(gi, _):
        g = wid * 4 + gi
        base = g * 8

        def fill(j, u):
            stage[pl.ds(j * 16, 16)] = jnp.full((16,), _SENT, jnp.float32)
            return u
        lax.fori_loop(0, 8 * _W // 16, fill, 0)

        pltpu.sync_copy(tau_hbm.at[pl.ds(base, 8)], tauv8)
        tvals = [tauv8[x, :].reshape(16) for x in range(8)]

        def compact_rows(cb, nvr, offs):
            def vbody(j, os_):
                new = []
                for x in range(8):
                    v = cb[x, pl.ds(j * 16, 16)].reshape(16)
                    m = v >= tvals[x]
                    cs = plsc.cumsum(m.astype(jnp.int32))
                    idx = (x * _W + os_[x] - 1) + cs
                    plsc.store_scatter(stage, [idx], v, mask=m)
                    new.append(jnp.minimum(os_[x] + cs[15], _W - 16))
                return tuple(new)
            return lax.fori_loop(0, nvr, vbody, offs)

        def issue(c, buf, sem):
            return pltpu.async_copy(
                sims_hbm.at[pl.ds(base, 8), pl.ds(c * _CHL, _CHL)], buf, sem)

        def wait(buf, sem):
            pltpu.make_async_copy(
                sims_hbm.at[pl.ds(0, 8), pl.ds(0, _CHL)], buf, sem).wait()

        issue(0, cbuf, sema)

        def dbl_body(c2, offs):
            issue(2 * c2 + 1, cbufb, semb)
            wait(cbuf, sema)
            offs = compact_rows(cbuf, _CHL // 16, offs)
            issue(2 * c2 + 2, cbuf, sema)
            wait(cbufb, semb)
            return compact_rows(cbufb, _CHL // 16, offs)

        offs = lax.fori_loop(0, (_NFC - 1) // 2, dbl_body, (jnp.int32(0),) * 8)
        wait(cbuf, sema)
        compact_rows(cbuf, _CHL // 16, offs)

        for x in range(8):
            pltpu.sync_copy(stage.at[pl.ds(x * _W, _W)],
                            out_hbm.at[pl.ds((base + x) * _W, _W)])
        return _

    lax.fori_loop(0, 4, group_body, 0)


def _compact(sims, tau16):
    mesh = plsc.VectorSubcoreMesh(core_axis_name="c", subcore_axis_name="s")
    f = functools.partial(
        pl.kernel,
        out_type=jax.ShapeDtypeStruct((_B * _W,), jnp.float32),
        mesh=mesh,
        compiler_params=pltpu.CompilerParams(needs_layout_passes=False),
        scratch_types=[
            pltpu.VMEM((8, _CHL), jnp.float32),
            pltpu.VMEM((8, _CHL), jnp.float32),
            pltpu.VMEM((8 * _W,), jnp.float32),
            pltpu.VMEM((8, 16), jnp.float32),
            pltpu.SemaphoreType.DMA,
            pltpu.SemaphoreType.DMA,
        ],
    )(_sc_compact)
    return f(sims, tau16).reshape(_B, _W)


_RB = 128


def _final_kernel(buf_ref, sims_ref, tail_ref, pidx_ref, c_ref, e_ref, l_ref):
    tail = tail_ref[...]
    tcol = lax.broadcasted_iota(jnp.int32, (_RB, 128), 1)
    tail = jnp.where(tcol < _TAILC, tail, _SENT)
    s = jnp.concatenate([buf_ref[...], tail], axis=1)
    p = pidx_ref[...]
    key = _keyify(s)

    lo = jnp.zeros((_RB, 1), jnp.uint32)
    for bit in range(31, -1, -1):
        cand = lo | np.uint32(1 << bit)
        cnt = jnp.sum((key >= cand).astype(jnp.int32), axis=1, keepdims=True)
        lo = jnp.where(cnt >= _NBG, cand, lo)
    taukey = lo

    tgt = p + 1
    lo2 = jnp.zeros((_RB, 1), jnp.uint32)
    for bit in range(31, -1, -1):
        cand = lo2 | np.uint32(1 << bit)
        cnt = jnp.sum((key >= cand).astype(jnp.int32), axis=1, keepdims=True)
        cnt = jnp.minimum(cnt, _NBG)
        lo2 = jnp.where(cnt >= tgt, cand, lo2)

    tau = _unkeyify(taukey)
    vp = _unkeyify(lo2)
    gt = key > taukey
    kgt = jnp.sum(gt.astype(jnp.int32), axis=1, keepdims=True)
    npad = (_NBG - kgt).astype(jnp.float32)
    sm = jnp.where(gt, s, _SENT)

    s4 = sims_ref[...]
    iot = lax.broadcasted_iota(jnp.int32, (_RB, _NBG), 1)
    ps = jnp.sum(jnp.where(iot == p, s4, 0.0), axis=1, keepdims=True)

    def entropy_at(t):
        F = jnp.exp(sm / t)
        Ft = jnp.exp(tau / t)
        Fp = jnp.exp(vp / t)
        Z = jnp.sum(F, axis=1, keepdims=True) + npad * Ft - Fp
        S = (jnp.sum(sm * F, axis=1, keepdims=True)
             + npad * tau * Ft - vp * Fp)
        return jnp.log(Z) - S / (t * Z)

    t = jnp.full((_RB, 1), 5.0, jnp.float32)
    scale = 2.5
    for _ in range(13):
        ent = entropy_at(t)
        ind = 2.0 * (ent < _TARGET_H).astype(jnp.float32) - 1.0
        t = t + scale * ind
        scale = scale * 0.5
    ent = entropy_at(t)

    den = (jnp.sum(jnp.exp(sm / t - 1.0 / t), axis=1, keepdims=True)
           + npad * jnp.exp(tau / t - 1.0 / t))
    cp = jnp.exp(ps / t - 1.0 / t) / den
    c_ref[...] = t
    e_ref[...] = ent
    l_ref[...] = jnp.log(cp + 1e-7)


def _finalize(buf, sims, point_indices):
    nrb = _B // _RB
    return pl.pallas_call(
        _final_kernel,
        grid=(nrb,),
        in_specs=[
            pl.BlockSpec((_RB, _W), lambda i: (i, 0)),
            pl.BlockSpec((_RB, _NBG), lambda i: (i, 0)),
            pl.BlockSpec((_RB, 128), lambda i: (i, _KSC // 128)),
            pl.BlockSpec((_RB, 1), lambda i: (i, 0)),
        ],
        out_specs=[
            pl.BlockSpec((_RB, 1), lambda i: (i, 0)),
            pl.BlockSpec((_RB, 1), lambda i: (i, 0)),
            pl.BlockSpec((_RB, 1), lambda i: (i, 0)),
        ],
        out_shape=[
            jax.ShapeDtypeStruct((_B, 1), jnp.float32),
            jax.ShapeDtypeStruct((_B, 1), jnp.float32),
            jax.ShapeDtypeStruct((_B, 1), jnp.float32),
        ],
    )(buf, sims, sims, point_indices.reshape(_B, 1).astype(jnp.int32))


def kernel(points, point_indices, memory_bank):
    sims, sample = _compute_sims(points, memory_bank)
    tau16 = _sample_bisect(sample)
    buf = _compact(sims, tau16)
    centers, entropy, logterm = _finalize(buf, sims, point_indices)
    loss = -jnp.mean(logterm)
    return loss, sims, jnp.mean(centers), jnp.mean(entropy)
